# Initial kernel scaffold; baseline (speedup 1.0000x reference)
import jax
import jax.numpy as jnp
from jax.experimental import pallas as pl


def kernel(input, mask, weight, bias):
    raise NotImplementedError("write your pallas kernel here")


# speedup vs baseline: 1.2013x; 1.2013x over previous
import jax
import jax.numpy as jnp
from jax.experimental import pallas as pl
from jax.experimental.pallas import tpu as pltpu

_TW = 6144


def _body(x_ref, m_ref, w_ref, b_ref, o_ref):
    m = m_ref[0]
    x = x_ref[0]
    nl = m.shape[0]
    iota = jax.lax.broadcasted_iota(jnp.int32, m.shape, 0)
    mx = jnp.max(m, axis=0, keepdims=True)
    cand = jnp.where(m == mx, iota, nl)
    idx = jnp.min(cand, axis=0, keepdims=True)
    oh = (iota == idx).astype(jnp.float32)
    gw = jnp.dot(w_ref[...], oh, preferred_element_type=jnp.float32)
    gb = jnp.dot(b_ref[...], oh, preferred_element_type=jnp.float32)
    o_ref[0] = x * gw + gb


def kernel(input, mask, weight, bias):
    n, c, h, w = input.shape
    nl = mask.shape[1]
    s = h * w
    x2 = input.reshape(n, c, s)
    m2 = mask.reshape(n, nl, s)
    wt = weight.T
    bt = bias.T
    grid = (n, s // _TW)
    out = pl.pallas_call(
        _body,
        grid=grid,
        in_specs=[
            pl.BlockSpec((1, c, _TW), lambda i, j: (i, 0, j)),
            pl.BlockSpec((1, nl, _TW), lambda i, j: (i, 0, j)),
            pl.BlockSpec((c, nl), lambda i, j: (0, 0)),
            pl.BlockSpec((c, nl), lambda i, j: (0, 0)),
        ],
        out_specs=pl.BlockSpec((1, c, _TW), lambda i, j: (i, 0, j)),
        out_shape=jax.ShapeDtypeStruct((n, c, s), jnp.float32),
        compiler_params=pltpu.CompilerParams(
            dimension_semantics=("parallel", "parallel"),
        ),
    )(x2, m2, wt, bt)
    return out.reshape(n, c, h, w)
